# baseline (speedup 1.0000x reference)
import jax
import jax.numpy as jnp
from jax.experimental import pallas as pl

BATCH = 32
SEQ_LEN = 8192
N_ALLELES = 4
D_MODEL = 64
XR = BATCH * SEQ_LEN * N_ALLELES // 128
OR_ = BATCH * SEQ_LEN * D_MODEL // (16 * 128)
PR = 65536 * D_MODEL // (16 * 128)
R_TILE = 256


def _body(x_ref, w_ref, p_ref, o_ref):
    xb = x_ref[...]
    x2 = jnp.broadcast_to(xb[:, None, :], (R_TILE, 16, 128))
    k = jax.lax.broadcasted_iota(jnp.int32, (16, 128), 0)
    c = jax.lax.broadcasted_iota(jnp.int32, (16, 128), 1)
    mask = (c // 8 == k).astype(jnp.float32)
    x2m = x2 * mask[None, :, :]
    emb = jax.lax.dot_general(
        x2m, w_ref[...],
        dimension_numbers=(((2,), (0,)), ((), ())),
        preferred_element_type=jnp.float32,
    )
    o_ref[...] = emb + p_ref[...]


def kernel(x, allele_embedding, position_table):
    xv = x.reshape(XR, 128)
    pv = position_table.reshape(PR, 16, 128)
    l = jnp.arange(128)
    base = jnp.tile(allele_embedding, (128 // N_ALLELES, 128 // D_MODEL))
    w = base * ((l[:, None] // N_ALLELES % 2) == (l[None, :] // D_MODEL))
    out = pl.pallas_call(
        _body,
        grid=(OR_ // R_TILE,),
        in_specs=[
            pl.BlockSpec((R_TILE, 128), lambda g: (g, 0)),
            pl.BlockSpec((128, 128), lambda g: (0, 0)),
            pl.BlockSpec((R_TILE, 16, 128), lambda g: (0, 0, 0)),
        ],
        out_specs=pl.BlockSpec((R_TILE, 16, 128), lambda g: (g, 0, 0)),
        out_shape=jax.ShapeDtypeStruct((OR_, 16, 128), jnp.float32),
    )(xv, w, pv)
    return out.reshape(BATCH, SEQ_LEN, D_MODEL)


# speedup vs baseline: 1.5412x; 1.5412x over previous
import jax
import jax.numpy as jnp
from jax.experimental import pallas as pl

BATCH = 32
SEQ_LEN = 8192
N_ALLELES = 4
D_MODEL = 64


def _body(x_ref, a_ref, p_ref, o_ref):
    emb = jax.lax.dot_general(
        x_ref[0], a_ref[...],
        dimension_numbers=(((1,), (0,)), ((), ())),
        preferred_element_type=jnp.float32,
    )
    o_ref[0] = emb + p_ref[...]


def kernel(x, allele_embedding, position_table):
    return pl.pallas_call(
        _body,
        grid=(BATCH,),
        in_specs=[
            pl.BlockSpec((1, SEQ_LEN, N_ALLELES), lambda b: (b, 0, 0)),
            pl.BlockSpec((N_ALLELES, D_MODEL), lambda b: (0, 0)),
            pl.BlockSpec((SEQ_LEN, D_MODEL), lambda b: (0, 0)),
        ],
        out_specs=pl.BlockSpec((1, SEQ_LEN, D_MODEL), lambda b: (b, 0, 0)),
        out_shape=jax.ShapeDtypeStruct((BATCH, SEQ_LEN, D_MODEL), jnp.float32),
    )(x, allele_embedding, position_table)


# speedup vs baseline: 1.5428x; 1.0011x over previous
import jax
import jax.numpy as jnp
from jax.experimental import pallas as pl

BATCH = 32
SEQ_LEN = 8192
N_ALLELES = 4
D_MODEL = 64


def _body(x_ref, a_ref, p_ref, o_ref):
    for i in range(2):
        emb = jax.lax.dot_general(
            x_ref[i], a_ref[...],
            dimension_numbers=(((1,), (0,)), ((), ())),
            preferred_element_type=jnp.float32,
        )
        o_ref[i] = emb + p_ref[...]


def kernel(x, allele_embedding, position_table):
    return pl.pallas_call(
        _body,
        grid=(BATCH // 2,),
        in_specs=[
            pl.BlockSpec((2, SEQ_LEN, N_ALLELES), lambda b: (b, 0, 0)),
            pl.BlockSpec((N_ALLELES, D_MODEL), lambda b: (0, 0)),
            pl.BlockSpec((SEQ_LEN, D_MODEL), lambda b: (0, 0)),
        ],
        out_specs=pl.BlockSpec((2, SEQ_LEN, D_MODEL), lambda b: (b, 0, 0)),
        out_shape=jax.ShapeDtypeStruct((BATCH, SEQ_LEN, D_MODEL), jnp.float32),
    )(x, allele_embedding, position_table)
